# Initial kernel scaffold; baseline (speedup 1.0000x reference)
import jax
import jax.numpy as jnp
from jax.experimental import pallas as pl


def kernel(x, code_h_outer, gate_thresh, dec_thresh, gate_perm, dec_perm):
    raise NotImplementedError("write your pallas kernel here")


# speedup vs baseline: 4.1088x; 4.1088x over previous
import functools

import jax
import jax.numpy as jnp
from jax import lax
from jax.experimental import pallas as pl
from jax.experimental.pallas import tpu as pltpu
from jax.experimental.pallas import tpu_sc as plsc

B = 4096
L = 1024
E = 8
NCHK = 16
NCOMBO = 2 * E
INF_CRC = 100.0

ROW_TILE = 512
NUM_TILES = B // ROW_TILE

NC, NS, LANES = 2, 16, 16
NW = NC * NS
ROWS_PER_W = B // NW
RB = 16
NBLK = ROWS_PER_W // RB


def _hp_kernel(perm_ref, h_ref, out_ref):
    pr = perm_ref[0]
    jidx = lax.broadcasted_iota(jnp.int32, (L, L), 0)
    pt = (jidx == pr).astype(jnp.bfloat16)
    h = h_ref[...].astype(jnp.bfloat16)
    hp = lax.dot_general(h, pt, (((1,), (1,)), ((), ())),
                         preferred_element_type=jnp.float32)
    out_ref[0] = hp - 2.0 * jnp.floor(hp * 0.5)


def _route_kernel(x_ref, hp_ref, th_ref, combo_ref, tsel_ref):
    x = x_ref[...]
    crcs = []
    for c in range(NCOMBO):
        y = (x > th_ref[c]).astype(jnp.bfloat16)
        hc = hp_ref[c].astype(jnp.bfloat16)
        pre = lax.dot_general(hc, y, (((1,), (1,)), ((), ())),
                              preferred_element_type=jnp.float32)
        par = pre - 2.0 * jnp.floor(pre * 0.5)
        crcs.append(jnp.sum(par, axis=0, keepdims=True))
    crc_all = jnp.concatenate(crcs, axis=0)
    crc_g, crc_d = crc_all[:E], crc_all[E:]
    ridx = lax.broadcasted_iota(jnp.int32, (E, ROW_TILE), 0).astype(jnp.float32)
    encg_min = jnp.min(crc_g * 8.0 + ridx, axis=0, keepdims=True)
    min_g = jnp.floor(encg_min * 0.125)
    gidx = encg_min - 8.0 * min_g
    non_zero = min_g > 0.0
    to_dec = (crc_g == min_g) & non_zero
    crc_e = jnp.where(to_dec, crc_d, INF_CRC)
    ence_min = jnp.min(crc_e * 8.0 + ridx, axis=0, keepdims=True)
    eidx = ence_min - 8.0 * jnp.floor(ence_min * 0.125)
    combo = jnp.where(non_zero, eidx + 8.0, gidx)
    tsel = jnp.zeros_like(combo)
    for c in range(NCOMBO):
        tsel = jnp.where(combo == float(c), th_ref[c], tsel)
    combo_ref[0] = combo.astype(jnp.int32)
    tsel_ref[0] = tsel


def _decode_body(x_hbm, perms_hbm, combo_hbm, tsel_hbm, out_hbm,
                 perm_v, x_v, out_v, combo_v, tsel_v):
    wid = lax.axis_index("s") * NC + lax.axis_index("c")
    base = wid * ROWS_PER_W
    pltpu.sync_copy(perms_hbm, perm_v)
    pltpu.sync_copy(combo_hbm.at[pl.ds(base, ROWS_PER_W)], combo_v)
    pltpu.sync_copy(tsel_hbm.at[pl.ds(base, ROWS_PER_W)], tsel_v)
    lane = lax.broadcasted_iota(jnp.int32, (LANES,), 0)
    ones = jnp.ones((LANES,), jnp.float32)
    zeros = jnp.zeros((LANES,), jnp.float32)
    for blk in range(NBLK):
        r0 = base + blk * RB
        pltpu.sync_copy(x_hbm.at[pl.ds(r0, RB)], x_v)

        def row_body(r, _, blk=blk):
            idxsplat = jnp.full((LANES,), blk * RB, jnp.int32) + r
            csplat = plsc.load_gather(combo_v, [idxsplat])
            tsplat = plsc.load_gather(tsel_v, [idxsplat])
            rsplat = jnp.full((LANES,), 0, jnp.int32) + r

            def col_body(jc, _):
                colv = lane + jc * LANES
                pidx = plsc.load_gather(perm_v, [csplat, colv])
                xv = plsc.load_gather(x_v, [rsplat, pidx])
                plsc.store_scatter(out_v, [rsplat, colv],
                                   jnp.where(xv > tsplat, ones, zeros))
                return 0

            lax.fori_loop(0, L // LANES, col_body, 0)
            return 0

        lax.fori_loop(0, RB, row_body, 0)
        pltpu.sync_copy(out_v, out_hbm.at[pl.ds(r0, RB)])


@functools.cache
def _decode_kernel():
    mesh = plsc.VectorSubcoreMesh(core_axis_name="c", subcore_axis_name="s",
                                  num_cores=NC, num_subcores=NS)
    return pl.kernel(
        _decode_body,
        out_type=jax.ShapeDtypeStruct((B, L), jnp.float32),
        mesh=mesh,
        compiler_params=pltpu.CompilerParams(needs_layout_passes=False),
        scratch_types=[
            pltpu.VMEM((NCOMBO, L), jnp.int32),
            pltpu.VMEM((RB, L), jnp.float32),
            pltpu.VMEM((RB, L), jnp.float32),
            pltpu.VMEM((ROWS_PER_W,), jnp.int32),
            pltpu.VMEM((ROWS_PER_W,), jnp.float32),
        ],
    )


def kernel(x, code_h_outer, gate_thresh, dec_thresh, gate_perm, dec_perm):
    perms = jnp.concatenate([gate_perm, dec_perm], axis=0).astype(jnp.int32)
    thresh = jnp.concatenate([gate_thresh, dec_thresh], axis=0)

    hp = pl.pallas_call(
        _hp_kernel,
        grid=(NCOMBO,),
        in_specs=[
            pl.BlockSpec((1, 1, L), lambda i: (i, 0, 0)),
            pl.BlockSpec((NCHK, L), lambda i: (0, 0)),
        ],
        out_specs=pl.BlockSpec((1, NCHK, L), lambda i: (i, 0, 0)),
        out_shape=jax.ShapeDtypeStruct((NCOMBO, NCHK, L), jnp.float32),
    )(perms.reshape(NCOMBO, 1, L), code_h_outer)

    combo3, tsel3 = pl.pallas_call(
        _route_kernel,
        grid=(NUM_TILES,),
        in_specs=[
            pl.BlockSpec((ROW_TILE, L), lambda i: (i, 0)),
            pl.BlockSpec((NCOMBO, NCHK, L), lambda i: (0, 0, 0)),
            pl.BlockSpec(memory_space=pltpu.SMEM),
        ],
        out_specs=[
            pl.BlockSpec((1, 1, ROW_TILE), lambda i: (i, 0, 0)),
            pl.BlockSpec((1, 1, ROW_TILE), lambda i: (i, 0, 0)),
        ],
        out_shape=[
            jax.ShapeDtypeStruct((NUM_TILES, 1, ROW_TILE), jnp.int32),
            jax.ShapeDtypeStruct((NUM_TILES, 1, ROW_TILE), jnp.float32),
        ],
    )(x, hp, thresh)

    return _decode_kernel()(x, perms, combo3.reshape(B), tsel3.reshape(B))


# speedup vs baseline: 8.8627x; 2.1570x over previous
import functools

import jax
import jax.numpy as jnp
from jax import lax
from jax.experimental import pallas as pl
from jax.experimental.pallas import tpu as pltpu
from jax.experimental.pallas import tpu_sc as plsc

B = 4096
L = 1024
E = 8
NCHK = 16
NCOMBO = 2 * E
INF_CRC = 100.0

ROW_TILE = 512
NUM_TILES = B // ROW_TILE

NC, NS, LANES = 2, 16, 16
NW = NC * NS
ROWS_PER_W = B // NW
RB = 16
NBLK = ROWS_PER_W // RB


def _hp_kernel(perm_ref, h_ref, out_ref):
    pr = perm_ref[0]
    jidx = lax.broadcasted_iota(jnp.int32, (L, L), 0)
    pt = (jidx == pr).astype(jnp.bfloat16)
    h = h_ref[...].astype(jnp.bfloat16)
    hp = lax.dot_general(h, pt, (((1,), (1,)), ((), ())),
                         preferred_element_type=jnp.float32)
    out_ref[0] = hp - 2.0 * jnp.floor(hp * 0.5)


def _route_kernel(x_ref, hp_ref, th_ref, combo_ref, tsel_ref):
    x = x_ref[...]
    crcs = []
    for c in range(NCOMBO):
        y = (x > th_ref[c]).astype(jnp.bfloat16)
        hc = hp_ref[c].astype(jnp.bfloat16)
        pre = lax.dot_general(hc, y, (((1,), (1,)), ((), ())),
                              preferred_element_type=jnp.float32)
        par = pre - 2.0 * jnp.floor(pre * 0.5)
        crcs.append(jnp.sum(par, axis=0, keepdims=True))
    crc_all = jnp.concatenate(crcs, axis=0)
    crc_g, crc_d = crc_all[:E], crc_all[E:]
    ridx = lax.broadcasted_iota(jnp.int32, (E, ROW_TILE), 0).astype(jnp.float32)
    encg_min = jnp.min(crc_g * 8.0 + ridx, axis=0, keepdims=True)
    min_g = jnp.floor(encg_min * 0.125)
    gidx = encg_min - 8.0 * min_g
    non_zero = min_g > 0.0
    to_dec = (crc_g == min_g) & non_zero
    crc_e = jnp.where(to_dec, crc_d, INF_CRC)
    ence_min = jnp.min(crc_e * 8.0 + ridx, axis=0, keepdims=True)
    eidx = ence_min - 8.0 * jnp.floor(ence_min * 0.125)
    combo = jnp.where(non_zero, eidx + 8.0, gidx)
    tsel = jnp.zeros_like(combo)
    for c in range(NCOMBO):
        tsel = jnp.where(combo == float(c), th_ref[c], tsel)
    combo_ref[0] = combo.astype(jnp.int32)
    tsel_ref[0] = tsel


def _decode_body(x_hbm, perms_hbm, combo_hbm, tsel_hbm, out_hbm,
                 perm_v, x_v, out_v, combo_v, tsel_v,
                 sin0, sin1, sout0, sout1):
    wid = lax.axis_index("s") * NC + lax.axis_index("c")
    base = wid * ROWS_PER_W
    pltpu.sync_copy(perms_hbm, perm_v)
    pltpu.sync_copy(combo_hbm.at[pl.ds(base, ROWS_PER_W)], combo_v)
    pltpu.sync_copy(tsel_hbm.at[pl.ds(base, ROWS_PER_W)], tsel_v)
    lane = lax.broadcasted_iota(jnp.int32, (LANES,), 0)
    ones = jnp.ones((LANES,), jnp.float32)
    zeros = jnp.zeros((LANES,), jnp.float32)
    sin = [sin0, sin1]
    sout = [sout0, sout1]
    in_h = [None, None]
    out_h = [None, None]
    in_h[0] = pltpu.async_copy(x_hbm.at[pl.ds(base, RB)], x_v.at[0], sin[0])
    for blk in range(NBLK):
        cur = blk % 2
        if blk + 1 < NBLK:
            nxt = 1 - cur
            in_h[nxt] = pltpu.async_copy(
                x_hbm.at[pl.ds(base + (blk + 1) * RB, RB)], x_v.at[nxt], sin[nxt])
        in_h[cur].wait()
        if out_h[cur] is not None:
            out_h[cur].wait()
        x_cur = x_v.at[cur]
        o_cur = out_v.at[cur]

        def row_body(r, _, blk=blk, x_cur=x_cur, o_cur=o_cur):
            idxsplat = jnp.full((LANES,), blk * RB, jnp.int32) + r
            csplat = plsc.load_gather(combo_v, [idxsplat])
            tsplat = plsc.load_gather(tsel_v, [idxsplat])
            rsplat = jnp.full((LANES,), 0, jnp.int32) + r

            @plsc.parallel_loop(0, L // LANES, 1, unroll=8)
            def col_body(jc):
                colv = lane + jc * LANES
                pidx = plsc.load_gather(perm_v, [csplat, colv])
                xv = plsc.load_gather(x_cur, [rsplat, pidx])
                plsc.store_scatter(o_cur, [rsplat, colv],
                                   jnp.where(xv > tsplat, ones, zeros))

            return 0

        lax.fori_loop(0, RB, row_body, 0)
        out_h[cur] = pltpu.async_copy(
            o_cur, out_hbm.at[pl.ds(base + blk * RB, RB)], sout[cur])
    out_h[0].wait()
    out_h[1].wait()


@functools.cache
def _decode_kernel():
    mesh = plsc.VectorSubcoreMesh(core_axis_name="c", subcore_axis_name="s",
                                  num_cores=NC, num_subcores=NS)
    return pl.kernel(
        _decode_body,
        out_type=jax.ShapeDtypeStruct((B, L), jnp.float32),
        mesh=mesh,
        compiler_params=pltpu.CompilerParams(needs_layout_passes=False),
        scratch_types=[
            pltpu.VMEM((NCOMBO, L), jnp.int32),
            pltpu.VMEM((2, RB, L), jnp.float32),
            pltpu.VMEM((2, RB, L), jnp.float32),
            pltpu.VMEM((ROWS_PER_W,), jnp.int32),
            pltpu.VMEM((ROWS_PER_W,), jnp.float32),
            pltpu.SemaphoreType.DMA,
            pltpu.SemaphoreType.DMA,
            pltpu.SemaphoreType.DMA,
            pltpu.SemaphoreType.DMA,
        ],
    )


def kernel(x, code_h_outer, gate_thresh, dec_thresh, gate_perm, dec_perm):
    perms = jnp.concatenate([gate_perm, dec_perm], axis=0).astype(jnp.int32)
    thresh = jnp.concatenate([gate_thresh, dec_thresh], axis=0)

    hp = pl.pallas_call(
        _hp_kernel,
        grid=(NCOMBO,),
        in_specs=[
            pl.BlockSpec((1, 1, L), lambda i: (i, 0, 0)),
            pl.BlockSpec((NCHK, L), lambda i: (0, 0)),
        ],
        out_specs=pl.BlockSpec((1, NCHK, L), lambda i: (i, 0, 0)),
        out_shape=jax.ShapeDtypeStruct((NCOMBO, NCHK, L), jnp.float32),
    )(perms.reshape(NCOMBO, 1, L), code_h_outer)

    combo3, tsel3 = pl.pallas_call(
        _route_kernel,
        grid=(NUM_TILES,),
        in_specs=[
            pl.BlockSpec((ROW_TILE, L), lambda i: (i, 0)),
            pl.BlockSpec((NCOMBO, NCHK, L), lambda i: (0, 0, 0)),
            pl.BlockSpec(memory_space=pltpu.SMEM),
        ],
        out_specs=[
            pl.BlockSpec((1, 1, ROW_TILE), lambda i: (i, 0, 0)),
            pl.BlockSpec((1, 1, ROW_TILE), lambda i: (i, 0, 0)),
        ],
        out_shape=[
            jax.ShapeDtypeStruct((NUM_TILES, 1, ROW_TILE), jnp.int32),
            jax.ShapeDtypeStruct((NUM_TILES, 1, ROW_TILE), jnp.float32),
        ],
    )(x, hp, thresh)

    return _decode_kernel()(x, perms, combo3.reshape(B), tsel3.reshape(B))
